# baseline (speedup 1.0000x reference)
import functools

import jax
import jax.numpy as jnp
from jax import lax
from jax.experimental import pallas as pl
from jax.experimental.pallas import tpu as pltpu
from jax.experimental.pallas import tpu_sc as plsc

EPS = 1e-5
MIN_NORM = 1e-10

CB = 80


def _pre_body(x_ref, w_ref, h_ref):
    x = x_ref[...]
    n = jnp.maximum(jnp.sqrt(jnp.sum(x * x, axis=-1, keepdims=True)), MIN_NORM)
    n_c = jnp.clip(n, MIN_NORM, 1.0 - EPS)
    at = 0.5 * jnp.log((1.0 + n_c) / (1.0 - n_c))
    t = at * x / n
    h_ref[...] = jnp.dot(t, w_ref[...], preferred_element_type=jnp.float32)


def _post_body(p_ref, b_ref, o_ref):
    agg = jnp.sum(p_ref[...], axis=0)
    n = jnp.maximum(jnp.sqrt(jnp.sum(agg * agg, -1, keepdims=True)), MIN_NORM)
    em = jnp.tanh(n) * agg / n
    n2 = jnp.maximum(jnp.sqrt(jnp.sum(em * em, -1, keepdims=True)), MIN_NORM)
    x = em * jnp.minimum(1.0, (1.0 - EPS) / n2)
    b = b_ref[...]
    nb = jnp.maximum(jnp.sqrt(jnp.sum(b * b, -1, keepdims=True)), MIN_NORM)
    bt = jnp.tanh(nb) * b / nb
    nb2 = jnp.maximum(jnp.sqrt(jnp.sum(bt * bt, -1, keepdims=True)), MIN_NORM)
    y = bt * jnp.minimum(1.0, (1.0 - EPS) / nb2)
    x2 = jnp.sum(x * x, -1, keepdims=True)
    y2 = jnp.sum(y * y, -1, keepdims=True)
    xy = jnp.sum(x * y, -1, keepdims=True)
    num = (1.0 + 2.0 * xy + y2) * x + (1.0 - x2) * y
    den = 1.0 + 2.0 * xy + x2 * y2
    den = jnp.where(jnp.abs(den) < 1e-15, 1e-15, den)
    m = num / den
    n3 = jnp.maximum(jnp.sqrt(jnp.sum(m * m, -1, keepdims=True)), MIN_NORM)
    o_ref[...] = m * jnp.minimum(1.0, (1.0 - EPS) / n3)


def _make_sc_agg(N, D, E):
    info = plsc.get_sparse_core_info()
    NC, NS = info.num_cores, info.num_subcores
    NW = NC * NS
    assert E % (NW * CB) == 0
    chunks = E // (NW * CB)
    NPAD = -(-N // (NS * 8)) * (NS * 8)
    rows_per_sub = NPAD // NS
    mesh = plsc.VectorSubcoreMesh(core_axis_name="c", subcore_axis_name="s")

    REC = 2 * CB

    @functools.partial(
        pl.kernel,
        mesh=mesh,
        out_type=jax.ShapeDtypeStruct((NC, NPAD, D), jnp.float32),
        scratch_types=[
            pltpu.VMEM((REC,), jnp.int32),
            pltpu.VMEM((REC,), jnp.int32),
            pltpu.VMEM((CB,), jnp.float32),
            pltpu.VMEM((CB,), jnp.float32),
            pltpu.VMEM((CB, D), jnp.float32),
            pltpu.VMEM((CB, D), jnp.float32),
            pltpu.VMEM_SHARED((NPAD, D), jnp.float32),
            pltpu.SemaphoreType.DMA,
            pltpu.SemaphoreType.DMA,
        ],
    )
    def sc_agg(h_hbm, edata_hbm, val_hbm, zero_hbm, out_hbm,
               eb_a, eb_b, vb_a, vb_b, rows_a, rows_b, acc, sem_a, sem_b):
        c = lax.axis_index("c")
        s = lax.axis_index("s")
        wid = s * NC + c
        pltpu.sync_copy(zero_hbm.at[pl.ds(s * rows_per_sub, rows_per_sub)],
                        acc.at[pl.ds(s * rows_per_sub, rows_per_sub)])
        plsc.subcore_barrier()
        bc = wid * chunks

        def load_rec(g, eb, vb):
            pltpu.sync_copy(edata_hbm.at[pl.ds((bc + g) * REC, REC)], eb)
            pltpu.sync_copy(val_hbm.at[pl.ds((bc + g) * CB, CB)], vb)

        def fire_gather(eb, buf, sem):
            pltpu.async_copy(h_hbm.at[eb.at[pl.ds(CB, CB)]], buf, sem)

        def wait_gather(eb, buf, sem):
            pltpu.make_async_copy(
                h_hbm.at[eb.at[pl.ds(CB, CB)]], buf, sem).wait()

        def process(eb, vb, buf):
            def scale_body(q, carry2):
                vv = vb[pl.ds(q * 16, 16)]
                for l in range(16):
                    sv = vv[l]
                    for j in range(D // 16):
                        sl = (q * 16 + l, pl.ds(16 * j, 16))
                        buf[sl] = buf[sl] * sv
                return carry2

            lax.fori_loop(0, CB // 16, scale_body, 0)
            for q in range(CB // 16):
                idx16 = eb[pl.ds(q * 16, 16)]
                pltpu.sync_copy(buf.at[pl.ds(q * 16, 16)], acc.at[idx16],
                                add=True)

        load_rec(0, eb_a, vb_a)
        fire_gather(eb_a, rows_a, sem_a)

        def pair_body(k, carry):
            g0 = 2 * k
            load_rec(g0 + 1, eb_b, vb_b)
            fire_gather(eb_b, rows_b, sem_b)
            wait_gather(eb_a, rows_a, sem_a)
            process(eb_a, vb_a, rows_a)
            load_rec(g0 + 2, eb_a, vb_a)
            fire_gather(eb_a, rows_a, sem_a)
            wait_gather(eb_b, rows_b, sem_b)
            process(eb_b, vb_b, rows_b)
            return carry

        lax.fori_loop(0, chunks // 2, pair_body, 0)
        wait_gather(eb_a, rows_a, sem_a)
        process(eb_a, vb_a, rows_a)
        plsc.subcore_barrier()
        pltpu.sync_copy(acc.at[pl.ds(s * rows_per_sub, rows_per_sub)],
                        out_hbm.at[c, pl.ds(s * rows_per_sub, rows_per_sub)])

    return sc_agg, NC, NPAD


def kernel(inputs, edge_index, adj_values, weight, bias):
    N, D = inputs.shape
    E = adj_values.shape[0]
    BN = 2000
    assert N % BN == 0

    h = pl.pallas_call(
        _pre_body,
        grid=(N // BN,),
        in_specs=[
            pl.BlockSpec((BN, D), lambda i: (i, 0)),
            pl.BlockSpec((D, D), lambda i: (0, 0)),
        ],
        out_specs=pl.BlockSpec((BN, D), lambda i: (i, 0)),
        out_shape=jax.ShapeDtypeStruct((N, D), jnp.float32),
    )(inputs, weight)

    sc_agg, NC, NPAD = _make_sc_agg(N, D, E)
    ei = edge_index.astype(jnp.int32)
    edata = jnp.stack(
        [ei[0].reshape(-1, CB), ei[1].reshape(-1, CB)], axis=1).reshape(-1)
    zeros = jnp.zeros((NPAD, D), jnp.float32)
    parts = sc_agg(h, edata, adj_values, zeros)

    out = pl.pallas_call(
        _post_body,
        grid=(N // BN,),
        in_specs=[
            pl.BlockSpec((NC, BN, D), lambda i: (0, i, 0)),
            pl.BlockSpec((1, D), lambda i: (0, 0)),
        ],
        out_specs=pl.BlockSpec((BN, D), lambda i: (i, 0)),
        out_shape=jax.ShapeDtypeStruct((N, D), jnp.float32),
    )(parts, bias.reshape(1, D))
    return out


# speedup vs baseline: 1.1133x; 1.1133x over previous
import functools

import jax
import jax.numpy as jnp
from jax import lax
from jax.experimental import pallas as pl
from jax.experimental.pallas import tpu as pltpu
from jax.experimental.pallas import tpu_sc as plsc

EPS = 1e-5
MIN_NORM = 1e-10

CB = 80


def _pre_body(x_ref, w_ref, h_ref):
    x = x_ref[...]
    n = jnp.maximum(jnp.sqrt(jnp.sum(x * x, axis=-1, keepdims=True)), MIN_NORM)
    n_c = jnp.clip(n, MIN_NORM, 1.0 - EPS)
    at = 0.5 * jnp.log((1.0 + n_c) / (1.0 - n_c))
    t = at * x / n
    h_ref[...] = jnp.dot(t, w_ref[...], preferred_element_type=jnp.float32)


def _post_body(p_ref, b_ref, o_ref):
    agg = jnp.sum(p_ref[...], axis=0)
    n = jnp.maximum(jnp.sqrt(jnp.sum(agg * agg, -1, keepdims=True)), MIN_NORM)
    em = jnp.tanh(n) * agg / n
    n2 = jnp.maximum(jnp.sqrt(jnp.sum(em * em, -1, keepdims=True)), MIN_NORM)
    x = em * jnp.minimum(1.0, (1.0 - EPS) / n2)
    b = b_ref[...]
    nb = jnp.maximum(jnp.sqrt(jnp.sum(b * b, -1, keepdims=True)), MIN_NORM)
    bt = jnp.tanh(nb) * b / nb
    nb2 = jnp.maximum(jnp.sqrt(jnp.sum(bt * bt, -1, keepdims=True)), MIN_NORM)
    y = bt * jnp.minimum(1.0, (1.0 - EPS) / nb2)
    x2 = jnp.sum(x * x, -1, keepdims=True)
    y2 = jnp.sum(y * y, -1, keepdims=True)
    xy = jnp.sum(x * y, -1, keepdims=True)
    num = (1.0 + 2.0 * xy + y2) * x + (1.0 - x2) * y
    den = 1.0 + 2.0 * xy + x2 * y2
    den = jnp.where(jnp.abs(den) < 1e-15, 1e-15, den)
    m = num / den
    n3 = jnp.maximum(jnp.sqrt(jnp.sum(m * m, -1, keepdims=True)), MIN_NORM)
    o_ref[...] = m * jnp.minimum(1.0, (1.0 - EPS) / n3)


def _make_sc_agg(N, D, E):
    info = plsc.get_sparse_core_info()
    NC, NS = info.num_cores, info.num_subcores
    NW = NC * NS
    assert E % (NW * CB) == 0
    chunks = E // (NW * CB)
    NPAD = -(-N // (NS * 8)) * (NS * 8)
    rows_per_sub = NPAD // NS
    mesh = plsc.VectorSubcoreMesh(core_axis_name="c", subcore_axis_name="s")

    REC = 2 * CB

    @functools.partial(
        pl.kernel,
        mesh=mesh,
        out_type=jax.ShapeDtypeStruct((NC, NPAD, D), jnp.float32),
        scratch_types=[
            pltpu.VMEM((REC,), jnp.int32),
            pltpu.VMEM((REC,), jnp.int32),
            pltpu.VMEM((CB,), jnp.float32),
            pltpu.VMEM((CB,), jnp.float32),
            pltpu.VMEM((CB, D), jnp.float32),
            pltpu.VMEM((CB, D), jnp.float32),
            pltpu.VMEM_SHARED((NPAD, D), jnp.float32),
            pltpu.SemaphoreType.DMA,
            pltpu.SemaphoreType.DMA,
            pltpu.SemaphoreType.DMA,
            pltpu.SemaphoreType.DMA,
        ],
    )
    def sc_agg(h_hbm, edata_hbm, val_hbm, zero_hbm, out_hbm,
               eb_a, eb_b, vb_a, vb_b, rows_a, rows_b, acc,
               sem_a, sem_b, sem_sa, sem_sb):
        c = lax.axis_index("c")
        s = lax.axis_index("s")
        wid = s * NC + c
        pltpu.sync_copy(zero_hbm.at[pl.ds(s * rows_per_sub, rows_per_sub)],
                        acc.at[pl.ds(s * rows_per_sub, rows_per_sub)])
        plsc.subcore_barrier()
        bc = wid * chunks

        def load_rec(g, eb, vb):
            pltpu.sync_copy(edata_hbm.at[pl.ds((bc + g) * REC, REC)], eb)
            pltpu.sync_copy(val_hbm.at[pl.ds((bc + g) * CB, CB)], vb)

        def fire_gather(eb, buf, sem):
            pltpu.async_copy(h_hbm.at[eb.at[pl.ds(CB, CB)]], buf, sem)

        def wait_gather(eb, buf, sem):
            pltpu.make_async_copy(
                h_hbm.at[eb.at[pl.ds(CB, CB)]], buf, sem).wait()

        def scale(vb, buf):
            def scale_body(q, carry2):
                vv = vb[pl.ds(q * 16, 16)]
                for l in range(16):
                    sv = vv[l]
                    for j in range(D // 16):
                        sl = (q * 16 + l, pl.ds(16 * j, 16))
                        buf[sl] = buf[sl] * sv
                return carry2

            lax.fori_loop(0, CB // 16, scale_body, 0)

        def fire_scatter(eb, buf, sem):
            return [
                pltpu.async_copy(buf.at[pl.ds(q * 16, 16)],
                                 acc.at[eb[pl.ds(q * 16, 16)]], sem,
                                 add=True)
                for q in range(CB // 16)
            ]

        def drain(descs):
            for dsc in descs:
                dsc.wait()

        load_rec(0, eb_a, vb_a)
        fire_gather(eb_a, rows_a, sem_a)
        load_rec(1, eb_b, vb_b)
        fire_gather(eb_b, rows_b, sem_b)

        def pair_body(k, carry):
            g0 = 2 * k
            wait_gather(eb_a, rows_a, sem_a)
            scale(vb_a, rows_a)
            d_a = fire_scatter(eb_a, rows_a, sem_sa)
            wait_gather(eb_b, rows_b, sem_b)
            scale(vb_b, rows_b)
            d_b = fire_scatter(eb_b, rows_b, sem_sb)
            drain(d_a)
            load_rec(g0 + 2, eb_a, vb_a)
            fire_gather(eb_a, rows_a, sem_a)
            drain(d_b)

            @pl.when(g0 + 3 < chunks)
            def _():
                load_rec(g0 + 3, eb_b, vb_b)
                fire_gather(eb_b, rows_b, sem_b)

            return carry

        lax.fori_loop(0, chunks // 2, pair_body, 0)
        wait_gather(eb_a, rows_a, sem_a)
        scale(vb_a, rows_a)
        drain(fire_scatter(eb_a, rows_a, sem_sa))
        plsc.subcore_barrier()
        pltpu.sync_copy(acc.at[pl.ds(s * rows_per_sub, rows_per_sub)],
                        out_hbm.at[c, pl.ds(s * rows_per_sub, rows_per_sub)])

    return sc_agg, NC, NPAD


def kernel(inputs, edge_index, adj_values, weight, bias):
    N, D = inputs.shape
    E = adj_values.shape[0]
    BN = 2000
    assert N % BN == 0

    h = pl.pallas_call(
        _pre_body,
        grid=(N // BN,),
        in_specs=[
            pl.BlockSpec((BN, D), lambda i: (i, 0)),
            pl.BlockSpec((D, D), lambda i: (0, 0)),
        ],
        out_specs=pl.BlockSpec((BN, D), lambda i: (i, 0)),
        out_shape=jax.ShapeDtypeStruct((N, D), jnp.float32),
    )(inputs, weight)

    sc_agg, NC, NPAD = _make_sc_agg(N, D, E)
    ei = edge_index.astype(jnp.int32)
    edata = jnp.stack(
        [ei[0].reshape(-1, CB), ei[1].reshape(-1, CB)], axis=1).reshape(-1)
    zeros = jnp.zeros((NPAD, D), jnp.float32)
    parts = sc_agg(h, edata, adj_values, zeros)

    out = pl.pallas_call(
        _post_body,
        grid=(N // BN,),
        in_specs=[
            pl.BlockSpec((NC, BN, D), lambda i: (0, i, 0)),
            pl.BlockSpec((1, D), lambda i: (0, 0)),
        ],
        out_specs=pl.BlockSpec((BN, D), lambda i: (i, 0)),
        out_shape=jax.ShapeDtypeStruct((N, D), jnp.float32),
    )(parts, bias.reshape(1, D))
    return out


# speedup vs baseline: 1.4357x; 1.2897x over previous
import functools

import jax
import jax.numpy as jnp
from jax import lax
from jax.experimental import pallas as pl
from jax.experimental.pallas import tpu as pltpu
from jax.experimental.pallas import tpu_sc as plsc

EPS = 1e-5
MIN_NORM = 1e-10

CB = 80


def _pre_body(x_ref, w_ref, h_ref):
    x = x_ref[...]
    n = jnp.maximum(jnp.sqrt(jnp.sum(x * x, axis=-1, keepdims=True)), MIN_NORM)
    n_c = jnp.clip(n, MIN_NORM, 1.0 - EPS)
    at = 0.5 * jnp.log((1.0 + n_c) / (1.0 - n_c))
    t = at * x / n
    h_ref[...] = jnp.dot(t, w_ref[...], preferred_element_type=jnp.float32)


def _post_body(p_ref, b_ref, o_ref):
    agg = jnp.sum(p_ref[...], axis=0)
    n = jnp.maximum(jnp.sqrt(jnp.sum(agg * agg, -1, keepdims=True)), MIN_NORM)
    em = jnp.tanh(n) * agg / n
    n2 = jnp.maximum(jnp.sqrt(jnp.sum(em * em, -1, keepdims=True)), MIN_NORM)
    x = em * jnp.minimum(1.0, (1.0 - EPS) / n2)
    b = b_ref[...]
    nb = jnp.maximum(jnp.sqrt(jnp.sum(b * b, -1, keepdims=True)), MIN_NORM)
    bt = jnp.tanh(nb) * b / nb
    nb2 = jnp.maximum(jnp.sqrt(jnp.sum(bt * bt, -1, keepdims=True)), MIN_NORM)
    y = bt * jnp.minimum(1.0, (1.0 - EPS) / nb2)
    x2 = jnp.sum(x * x, -1, keepdims=True)
    y2 = jnp.sum(y * y, -1, keepdims=True)
    xy = jnp.sum(x * y, -1, keepdims=True)
    num = (1.0 + 2.0 * xy + y2) * x + (1.0 - x2) * y
    den = 1.0 + 2.0 * xy + x2 * y2
    den = jnp.where(jnp.abs(den) < 1e-15, 1e-15, den)
    m = num / den
    n3 = jnp.maximum(jnp.sqrt(jnp.sum(m * m, -1, keepdims=True)), MIN_NORM)
    o_ref[...] = m * jnp.minimum(1.0, (1.0 - EPS) / n3)


def _make_sc_agg(N, D, E):
    info = plsc.get_sparse_core_info()
    NC, NS = info.num_cores, info.num_subcores
    NW = NC * NS
    assert E % (NW * CB) == 0
    chunks = E // (NW * CB)
    rows_per_sub = (N // NS) // 8 * 8
    mesh = plsc.VectorSubcoreMesh(core_axis_name="c", subcore_axis_name="s")

    REC = 2 * CB

    @functools.partial(
        pl.kernel,
        mesh=mesh,
        out_type=jax.ShapeDtypeStruct((NC, N, D), jnp.float32),
        scratch_types=[
            pltpu.VMEM((chunks * REC,), jnp.int32),
            pltpu.VMEM((chunks * CB,), jnp.float32),
            pltpu.VMEM((CB, D), jnp.float32),
            pltpu.VMEM((CB, D), jnp.float32),
            pltpu.VMEM_SHARED((N, D), jnp.float32),
            pltpu.SemaphoreType.DMA,
            pltpu.SemaphoreType.DMA,
            pltpu.SemaphoreType.DMA,
            pltpu.SemaphoreType.DMA,
        ],
    )
    def sc_agg(h_hbm, edata_hbm, val_hbm, zero_hbm, out_hbm,
               eb, vb, rows_a, rows_b, acc,
               sem_a, sem_b, sem_sa, sem_sb):
        c = lax.axis_index("c")
        s = lax.axis_index("s")
        wid = s * NC + c
        R0 = rows_per_sub
        RL = N - (NS - 1) * R0

        @pl.when(s < NS - 1)
        def _():
            pltpu.sync_copy(zero_hbm.at[pl.ds(s * R0, R0)],
                            acc.at[pl.ds(s * R0, R0)])

        @pl.when(s == NS - 1)
        def _():
            pltpu.sync_copy(zero_hbm.at[pl.ds((NS - 1) * R0, RL)],
                            acc.at[pl.ds((NS - 1) * R0, RL)])

        bc = wid * chunks
        pltpu.sync_copy(edata_hbm.at[pl.ds(bc * REC, chunks * REC)], eb)
        pltpu.sync_copy(val_hbm.at[pl.ds(bc * CB, chunks * CB)], vb)
        plsc.subcore_barrier()

        def fire_gather(g, buf, sem):
            pltpu.async_copy(
                h_hbm.at[eb.at[pl.ds(g * REC + CB, CB)]], buf, sem)

        def wait_gather(buf, sem):
            pltpu.make_async_copy(
                h_hbm.at[eb.at[pl.ds(CB, CB)]], buf, sem).wait()

        def scale(g, buf):
            def scale_body(q, carry2):
                vv = vb[pl.ds(g * CB + q * 16, 16)]
                for l in range(16):
                    sv = vv[l]
                    for j in range(D // 16):
                        sl = (q * 16 + l, pl.ds(16 * j, 16))
                        buf[sl] = buf[sl] * sv
                return carry2

            lax.fori_loop(0, CB // 16, scale_body, 0)

        def fire_scatter(g, buf, sem):
            return [
                pltpu.async_copy(buf.at[pl.ds(q * 16, 16)],
                                 acc.at[eb[pl.ds(g * REC + q * 16, 16)]],
                                 sem, add=True)
                for q in range(CB // 16)
            ]

        def drain(descs):
            for dsc in descs:
                dsc.wait()

        fire_gather(0, rows_a, sem_a)
        fire_gather(1, rows_b, sem_b)

        def pair_body(k, carry):
            g0 = 2 * k
            wait_gather(rows_a, sem_a)
            scale(g0, rows_a)
            d_a = fire_scatter(g0, rows_a, sem_sa)
            wait_gather(rows_b, sem_b)
            scale(g0 + 1, rows_b)
            d_b = fire_scatter(g0 + 1, rows_b, sem_sb)
            drain(d_a)
            fire_gather(g0 + 2, rows_a, sem_a)
            drain(d_b)

            @pl.when(g0 + 3 < chunks)
            def _():
                fire_gather(g0 + 3, rows_b, sem_b)

            return carry

        lax.fori_loop(0, chunks // 2, pair_body, 0)
        wait_gather(rows_a, sem_a)
        scale(chunks - 1, rows_a)
        drain(fire_scatter(chunks - 1, rows_a, sem_sa))
        plsc.subcore_barrier()

        @pl.when(s < NS - 1)
        def _():
            pltpu.sync_copy(acc.at[pl.ds(s * R0, R0)],
                            out_hbm.at[c, pl.ds(s * R0, R0)])

        @pl.when(s == NS - 1)
        def _():
            pltpu.sync_copy(acc.at[pl.ds((NS - 1) * R0, RL)],
                            out_hbm.at[c, pl.ds((NS - 1) * R0, RL)])

    return sc_agg, NC


def kernel(inputs, edge_index, adj_values, weight, bias):
    N, D = inputs.shape
    E = adj_values.shape[0]
    BN = 2000
    assert N % BN == 0

    h = pl.pallas_call(
        _pre_body,
        grid=(N // BN,),
        in_specs=[
            pl.BlockSpec((BN, D), lambda i: (i, 0)),
            pl.BlockSpec((D, D), lambda i: (0, 0)),
        ],
        out_specs=pl.BlockSpec((BN, D), lambda i: (i, 0)),
        out_shape=jax.ShapeDtypeStruct((N, D), jnp.float32),
    )(inputs, weight)

    sc_agg, NC = _make_sc_agg(N, D, E)
    ei = edge_index.astype(jnp.int32)
    edata = jnp.stack(
        [ei[0].reshape(-1, CB), ei[1].reshape(-1, CB)], axis=1).reshape(-1)
    zeros = jnp.zeros((N, D), jnp.float32)
    parts = sc_agg(h, edata, adj_values, zeros)

    out = pl.pallas_call(
        _post_body,
        grid=(N // BN,),
        in_specs=[
            pl.BlockSpec((NC, BN, D), lambda i: (0, i, 0)),
            pl.BlockSpec((1, D), lambda i: (0, 0)),
        ],
        out_specs=pl.BlockSpec((BN, D), lambda i: (i, 0)),
        out_shape=jax.ShapeDtypeStruct((N, D), jnp.float32),
    )(parts, bias.reshape(1, D))
    return out


# speedup vs baseline: 1.6711x; 1.1639x over previous
import functools

import jax
import jax.numpy as jnp
from jax import lax
from jax.experimental import pallas as pl
from jax.experimental.pallas import tpu as pltpu
from jax.experimental.pallas import tpu_sc as plsc

EPS = 1e-5
MIN_NORM = 1e-10

CB = 80
NBUF = 3


def _pre_body(x_ref, w_ref, h_ref):
    x = x_ref[...]
    n = jnp.maximum(jnp.sqrt(jnp.sum(x * x, axis=-1, keepdims=True)), MIN_NORM)
    n_c = jnp.clip(n, MIN_NORM, 1.0 - EPS)
    at = 0.5 * jnp.log((1.0 + n_c) / (1.0 - n_c))
    t = at * x / n
    h_ref[...] = jnp.dot(t, w_ref[...], preferred_element_type=jnp.float32)


def _post_body(p_ref, b_ref, o_ref):
    agg = jnp.sum(p_ref[...], axis=0)
    n = jnp.maximum(jnp.sqrt(jnp.sum(agg * agg, -1, keepdims=True)), MIN_NORM)
    em = jnp.tanh(n) * agg / n
    n2 = jnp.maximum(jnp.sqrt(jnp.sum(em * em, -1, keepdims=True)), MIN_NORM)
    x = em * jnp.minimum(1.0, (1.0 - EPS) / n2)
    b = b_ref[...]
    nb = jnp.maximum(jnp.sqrt(jnp.sum(b * b, -1, keepdims=True)), MIN_NORM)
    bt = jnp.tanh(nb) * b / nb
    nb2 = jnp.maximum(jnp.sqrt(jnp.sum(bt * bt, -1, keepdims=True)), MIN_NORM)
    y = bt * jnp.minimum(1.0, (1.0 - EPS) / nb2)
    x2 = jnp.sum(x * x, -1, keepdims=True)
    y2 = jnp.sum(y * y, -1, keepdims=True)
    xy = jnp.sum(x * y, -1, keepdims=True)
    num = (1.0 + 2.0 * xy + y2) * x + (1.0 - x2) * y
    den = 1.0 + 2.0 * xy + x2 * y2
    den = jnp.where(jnp.abs(den) < 1e-15, 1e-15, den)
    m = num / den
    n3 = jnp.maximum(jnp.sqrt(jnp.sum(m * m, -1, keepdims=True)), MIN_NORM)
    o_ref[...] = m * jnp.minimum(1.0, (1.0 - EPS) / n3)


def _make_sc_agg(N, D, E):
    info = plsc.get_sparse_core_info()
    NC, NS = info.num_cores, info.num_subcores
    NW = NC * NS
    assert E % (NW * CB) == 0
    chunks = E // (NW * CB)
    rows_per_sub = (N // NS) // 8 * 8
    mesh = plsc.VectorSubcoreMesh(core_axis_name="c", subcore_axis_name="s")
    EW = chunks * CB

    PCH0 = (chunks + 1) // 2
    passes = [(0, PCH0), (PCH0, chunks - PCH0)]

    @functools.partial(
        pl.kernel,
        mesh=mesh,
        out_type=jax.ShapeDtypeStruct((NC, N, D), jnp.float32),
        scratch_types=[
            pltpu.VMEM((PCH0 * CB,), jnp.int32),
            pltpu.VMEM((PCH0 * CB,), jnp.float32),
            [pltpu.VMEM((CB,), jnp.int32)] * NBUF,
            [pltpu.VMEM((CB, D), jnp.float32)] * NBUF,
            [pltpu.SemaphoreType.DMA] * NBUF,
            [pltpu.SemaphoreType.DMA] * NBUF,
            pltpu.VMEM_SHARED((N, D), jnp.float32),
        ],
    )
    def sc_agg(h_hbm, edata_hbm, val_hbm, zero_hbm, out_hbm,
               eb, vb, idxs, rows, gsems, ssems, acc):
        c = lax.axis_index("c")
        s = lax.axis_index("s")
        wid = s * NC + c
        R0 = rows_per_sub
        RL = N - (NS - 1) * R0

        @pl.when(s < NS - 1)
        def _():
            pltpu.sync_copy(zero_hbm.at[pl.ds(s * R0, R0)],
                            acc.at[pl.ds(s * R0, R0)])

        @pl.when(s == NS - 1)
        def _():
            pltpu.sync_copy(zero_hbm.at[pl.ds((NS - 1) * R0, RL)],
                            acc.at[pl.ds((NS - 1) * R0, RL)])

        plsc.subcore_barrier()

        def fire_gather(g, b):
            for q in range(CB // 16):
                p = eb[pl.ds(g * CB + q * 16, 16)]
                idxs[b][pl.ds(q * 16, 16)] = p >> 16
            pltpu.async_copy(h_hbm.at[idxs[b]], rows[b], gsems[b])

        def wait_gather(b):
            pltpu.make_async_copy(h_hbm.at[idxs[b]], rows[b],
                                  gsems[b]).wait()

        def scale(g, b):
            buf = rows[b]

            def scale_body(q, carry2):
                vv = vb[pl.ds(g * CB + q * 16, 16)]
                for l in range(16):
                    sv = vv[l]
                    for j in range(D // 16):
                        sl = (q * 16 + l, pl.ds(16 * j, 16))
                        buf[sl] = buf[sl] * sv
                return carry2

            lax.fori_loop(0, CB // 16, scale_body, 0)

        def fire_scatter(g, b):
            return [
                pltpu.async_copy(
                    rows[b].at[pl.ds(q * 16, 16)],
                    acc.at[eb[pl.ds(g * CB + q * 16, 16)] & 0xFFFF],
                    ssems[b], add=True)
                for q in range(CB // 16)
            ]

        def drain(descs):
            for dsc in descs:
                dsc.wait()

        for pbase, pch in passes:
            ebase = (wid * chunks + pbase) * CB
            pltpu.sync_copy(edata_hbm.at[pl.ds(ebase, pch * CB)],
                            eb.at[pl.ds(0, pch * CB)])
            pltpu.sync_copy(val_hbm.at[pl.ds(ebase, pch * CB)],
                            vb.at[pl.ds(0, pch * CB)])
            for b in range(NBUF):
                fire_gather(b, b)

            def loop_body(k, carry):
                g0 = NBUF * k
                descs = []
                for b in range(NBUF):
                    wait_gather(b)
                    scale(g0 + b, b)
                    descs.append(fire_scatter(g0 + b, b))
                for b in range(NBUF):
                    drain(descs[b])
                    g = g0 + NBUF + b

                    @pl.when(g < pch)
                    def _():
                        fire_gather(g, b)
                return carry

            iters = (pch - NBUF) // NBUF + 1
            lax.fori_loop(0, iters, loop_body, 0)
            tail = pch - NBUF * iters
            for b in range(tail):
                wait_gather(b)
                scale(NBUF * iters + b, b)
                drain(fire_scatter(NBUF * iters + b, b))
        plsc.subcore_barrier()

        @pl.when(s < NS - 1)
        def _():
            pltpu.sync_copy(acc.at[pl.ds(s * R0, R0)],
                            out_hbm.at[c, pl.ds(s * R0, R0)])

        @pl.when(s == NS - 1)
        def _():
            pltpu.sync_copy(acc.at[pl.ds((NS - 1) * R0, RL)],
                            out_hbm.at[c, pl.ds((NS - 1) * R0, RL)])

    return sc_agg, NC


def kernel(inputs, edge_index, adj_values, weight, bias):
    N, D = inputs.shape
    E = adj_values.shape[0]
    BN = 2000
    assert N % BN == 0

    h = pl.pallas_call(
        _pre_body,
        grid=(N // BN,),
        in_specs=[
            pl.BlockSpec((BN, D), lambda i: (i, 0)),
            pl.BlockSpec((D, D), lambda i: (0, 0)),
        ],
        out_specs=pl.BlockSpec((BN, D), lambda i: (i, 0)),
        out_shape=jax.ShapeDtypeStruct((N, D), jnp.float32),
    )(inputs, weight)

    sc_agg, NC = _make_sc_agg(N, D, E)
    ei = edge_index.astype(jnp.int32)
    edata = ei[0] | (ei[1] << 16)
    zeros = jnp.zeros((N, D), jnp.float32)
    parts = sc_agg(h, edata, adj_values, zeros)

    out = pl.pallas_call(
        _post_body,
        grid=(N // BN,),
        in_specs=[
            pl.BlockSpec((NC, BN, D), lambda i: (0, i, 0)),
            pl.BlockSpec((1, D), lambda i: (0, 0)),
        ],
        out_specs=pl.BlockSpec((BN, D), lambda i: (i, 0)),
        out_shape=jax.ShapeDtypeStruct((N, D), jnp.float32),
    )(parts, bias.reshape(1, D))
    return out
